# baseline (speedup 1.0000x reference)
import functools

import jax
import jax.numpy as jnp
from jax import lax
from jax.experimental import pallas as pl
from jax.experimental.pallas import tpu as pltpu
from jax.experimental.pallas import tpu_sc as plsc

B = 16384
EMB = 128
NUM_LOC = 58
NUM_AGE = 2

NC, NS, L = 2, 16, 16
NW = NC * NS
CPW = B // NW
G = 128
NCHUNK = CPW // G


@functools.lru_cache(maxsize=None)
def _build_sc_gather():
    @functools.partial(
        pl.kernel,
        mesh=plsc.VectorSubcoreMesh(core_axis_name="c", subcore_axis_name="s"),
        compiler_params=pltpu.CompilerParams(needs_layout_passes=False),
        out_type=jax.ShapeDtypeStruct((B, 2 * EMB), jnp.float32),
        scratch_types=[
            pltpu.VMEM((CPW,), jnp.int32),
            pltpu.VMEM((CPW,), jnp.int32),
            pltpu.VMEM((G, EMB), jnp.float32),
            pltpu.VMEM((G, EMB), jnp.float32),
            pltpu.VMEM((G, EMB), jnp.float32),
            pltpu.VMEM((G, EMB), jnp.float32),
            pltpu.VMEM((NUM_AGE, EMB), jnp.float32),
            pltpu.VMEM_SHARED((NUM_LOC, EMB), jnp.float32),
            pltpu.SemaphoreType.DMA,
            pltpu.SemaphoreType.DMA,
            pltpu.SemaphoreType.DMA,
            pltpu.SemaphoreType.DMA,
        ],
    )
    def _sc_gather(loc_idx_hbm, age_idx_hbm, loc_tab_hbm, age_tab_hbm,
                   out_hbm, loc_v, age_v, lbuf0, lbuf1, abuf0, abuf1,
                   age_tab_v, table_sh, gsem0, gsem1, wsem0, wsem1):
        lbufs = (lbuf0, lbuf1)
        abufs = (abuf0, abuf1)
        gsems = (gsem0, gsem1)
        wsems = (wsem0, wsem1)
        sid = lax.axis_index("s")
        wid = sid * NC + lax.axis_index("c")
        base = wid * CPW

        @pl.when(sid == 0)
        def _():
            pltpu.async_copy(loc_tab_hbm, table_sh, wsem0).wait()

        cp_l = pltpu.async_copy(loc_idx_hbm.at[pl.ds(base, CPW)], loc_v, gsem0)
        cp_a = pltpu.async_copy(age_idx_hbm.at[pl.ds(base, CPW)], age_v, gsem1)
        cp_t = pltpu.async_copy(age_tab_hbm, age_tab_v, wsem1)
        cp_l.wait()
        cp_a.wait()
        cp_t.wait()

        r0 = [age_tab_v[0, pl.ds(c * L, L)] for c in range(EMB // L)]
        r1 = [age_tab_v[1, pl.ds(c * L, L)] for c in range(EMB // L)]

        plsc.subcore_barrier()

        def start_gather(k, b):
            return pltpu.async_copy(
                table_sh.at[loc_v.at[pl.ds(k * G, G)]], lbufs[b], gsems[b])

        def compute_age(k, b):
            abuf = abufs[b]

            def body(g, carry):
                rl = g * L
                for j in range(L):
                    row = rl + j
                    spl = plsc.load_gather(
                        age_v, [jnp.full((L,), k * G + row, jnp.int32)])
                    mask = spl == 0
                    for c in range(EMB // L):
                        abuf[row, pl.ds(c * L, L)] = jnp.where(
                            mask, r0[c], r1[c])
                return carry

            lax.fori_loop(0, G // L, body, 0)

        gd = [None] * NCHUNK
        wdl = [None] * NCHUNK
        wda = [None] * NCHUNK
        gd[0] = start_gather(0, 0)
        for k in range(NCHUNK):
            b = k & 1
            if k + 1 < NCHUNK:
                if k >= 1:
                    wdl[k - 1].wait()
                gd[k + 1] = start_gather(k + 1, 1 - b)
            if k >= 2:
                wda[k - 2].wait()
            compute_age(k, b)
            gd[k].wait()
            row0_ = base + k * G
            wdl[k] = pltpu.async_copy(
                lbufs[b], out_hbm.at[pl.ds(row0_, G), pl.ds(0, EMB)],
                wsems[b])
            wda[k] = pltpu.async_copy(
                abufs[b], out_hbm.at[pl.ds(row0_, G), pl.ds(EMB, EMB)],
                wsems[b])
        for k in (NCHUNK - 2, NCHUNK - 1):
            wdl[k].wait()
            wda[k].wait()

    return _sc_gather


def kernel(location_idx, age_idx, location_table, age_table):
    return _build_sc_gather()(location_idx.astype(jnp.int32),
                              age_idx.astype(jnp.int32),
                              location_table, age_table)


# speedup vs baseline: 1.0218x; 1.0218x over previous
import functools

import jax
import jax.numpy as jnp
from jax import lax
from jax.experimental import pallas as pl
from jax.experimental.pallas import tpu as pltpu
from jax.experimental.pallas import tpu_sc as plsc

B = 16384
EMB = 128
NUM_LOC = 58
NUM_AGE = 2

NC, NS, L = 2, 16, 16
NW = NC * NS
CPW = B // NW
G = 128
NCHUNK = CPW // G


@functools.lru_cache(maxsize=None)
def _build_sc_gather():
    @functools.partial(
        pl.kernel,
        mesh=plsc.VectorSubcoreMesh(core_axis_name="c", subcore_axis_name="s"),
        compiler_params=pltpu.CompilerParams(
            needs_layout_passes=False,
            skip_device_barrier=True,
            disable_bounds_checks=True,
            disable_semaphore_checks=True,
        ),
        out_type=jax.ShapeDtypeStruct((B, 2 * EMB), jnp.float32),
        scratch_types=[
            pltpu.VMEM((CPW,), jnp.int32),
            pltpu.VMEM((CPW,), jnp.int32),
            pltpu.VMEM((G, EMB), jnp.float32),
            pltpu.VMEM((G, EMB), jnp.float32),
            pltpu.VMEM((G, EMB), jnp.float32),
            pltpu.VMEM((G, EMB), jnp.float32),
            pltpu.VMEM((G, EMB), jnp.float32),
            pltpu.VMEM((G, EMB), jnp.float32),
            pltpu.VMEM_SHARED((NUM_LOC + NUM_AGE, EMB), jnp.float32),
            pltpu.SemaphoreType.DMA,
            pltpu.SemaphoreType.DMA,
            pltpu.SemaphoreType.DMA,
            pltpu.SemaphoreType.DMA,
            pltpu.SemaphoreType.DMA,
            pltpu.SemaphoreType.DMA,
        ],
    )
    def _sc_gather(loc_idx_hbm, age_idx_hbm, loc_tab_hbm, age_tab_hbm,
                   out_hbm, loc_v, age_v, lbuf0, lbuf1, lbuf2,
                   abuf0, abuf1, abuf2, table_sh,
                   gsem0, gsem1, gsem2, wsem0, wsem1, wsem2):
        lbufs = (lbuf0, lbuf1, lbuf2)
        abufs = (abuf0, abuf1, abuf2)
        gsems = (gsem0, gsem1, gsem2)
        wsems = (wsem0, wsem1, wsem2)
        sid = lax.axis_index("s")
        wid = sid * NC + lax.axis_index("c")
        base = wid * CPW

        @pl.when(sid == 0)
        def _():
            tp_l = pltpu.async_copy(
                loc_tab_hbm, table_sh.at[pl.ds(0, NUM_LOC)], wsem0)
            tp_a = pltpu.async_copy(
                age_tab_hbm, table_sh.at[pl.ds(NUM_LOC, NUM_AGE)], wsem1)
            tp_l.wait()
            tp_a.wait()

        cp_l = pltpu.async_copy(loc_idx_hbm.at[pl.ds(base, CPW)], loc_v, gsem0)
        cp_a = pltpu.async_copy(age_idx_hbm.at[pl.ds(base, CPW)], age_v, gsem1)
        cp_l.wait()
        cp_a.wait()

        plsc.subcore_barrier()

        age_tab_sh = table_sh.at[pl.ds(NUM_LOC, NUM_AGE)]

        def start_gathers(k, b):
            return [
                pltpu.async_copy(
                    table_sh.at[loc_v.at[pl.ds(k * G, G)]], lbufs[b], gsems[b]),
                pltpu.async_copy(
                    age_tab_sh.at[age_v.at[pl.ds(k * G, G)]], abufs[b],
                    gsems[b]),
            ]

        gd = [None] * NCHUNK
        wd = [None] * NCHUNK
        gd[0] = start_gathers(0, 0)
        for k in range(NCHUNK):
            b = k % 3
            if k + 1 < NCHUNK:
                if k >= 2:
                    for d in wd[k - 2]:
                        d.wait()
                gd[k + 1] = start_gathers(k + 1, (k + 1) % 3)
            for d in gd[k]:
                d.wait()
            row0 = base + k * G
            wd[k] = [
                pltpu.async_copy(
                    lbufs[b], out_hbm.at[pl.ds(row0, G), pl.ds(0, EMB)],
                    wsems[b]),
                pltpu.async_copy(
                    abufs[b], out_hbm.at[pl.ds(row0, G), pl.ds(EMB, EMB)],
                    wsems[b]),
            ]
        for k in (NCHUNK - 2, NCHUNK - 1):
            for d in wd[k]:
                d.wait()

    return _sc_gather


def kernel(location_idx, age_idx, location_table, age_table):
    return _build_sc_gather()(location_idx.astype(jnp.int32),
                              age_idx.astype(jnp.int32),
                              location_table, age_table)
